# Initial kernel scaffold; baseline (speedup 1.0000x reference)
import jax
import jax.numpy as jnp
from jax.experimental import pallas as pl


def kernel(x, edge_index, W1l, b1l, W1r, W2l, b2l, W2r):
    raise NotImplementedError("write your pallas kernel here")


# speedup vs baseline: 6.8671x; 6.8671x over previous
import jax
import jax.numpy as jnp
from jax import lax
from jax.experimental import pallas as pl
from jax.experimental.pallas import tpu as pltpu
from jax.experimental.pallas import tpu_sc as plsc

_N_NODES = 10000
_N_EDGES = 320000
_N_PAD = 10240
_NC, _NS = 2, 16
_NW = _NC * _NS
_EPW = _N_EDGES // _NW
_CHUNK = 128
_NFULL = _EPW // _CHUNK
_TAIL = _EPW - _NFULL * _CHUNK
_RPT = _N_PAD // _NS
_RCH = _RPT // _CHUNK


def _make_sc_agg(d, with_cnt):
  mesh = plsc.VectorSubcoreMesh(core_axis_name="c", subcore_axis_name="s")
  out_type = [jax.ShapeDtypeStruct((_NC, _N_PAD, d), jnp.float32)]
  scratch = [
      pltpu.VMEM_SHARED((_N_PAD, d), jnp.float32),
      pltpu.VMEM((_CHUNK,), jnp.int32),
      pltpu.VMEM((_CHUNK,), jnp.int32),
      pltpu.VMEM((_CHUNK, d), jnp.float32),
      pltpu.VMEM((_TAIL,), jnp.int32),
      pltpu.VMEM((_TAIL,), jnp.int32),
      pltpu.VMEM((_TAIL, d), jnp.float32),
      pltpu.SemaphoreType.DMA,
  ]
  if with_cnt:
    out_type.append(jax.ShapeDtypeStruct((_NW, _N_PAD), jnp.float32))
    scratch += [pltpu.VMEM((_N_PAD,), jnp.float32)]

  def body(x_hbm, src_hbm, dst_hbm, *rest):
    cntl = cnt_hbm = None
    if with_cnt:
      (out_hbm, cnt_hbm, acc, sidx, didx, rows, sidx_t, didx_t, rows_t, sem,
       cntl) = rest
    else:
      out_hbm, acc, sidx, didx, rows, sidx_t, didx_t, rows_t, sem = rest
    cid = lax.axis_index("c")
    sid = lax.axis_index("s")
    wid = sid * _NC + cid
    row0 = sid * _RPT

    def _zrow(i, _):
      for j in range(d // 16):
        rows[i, pl.ds(j * 16, 16)] = jnp.zeros((16,), jnp.float32)
      return 0
    lax.fori_loop(0, _CHUNK, _zrow, 0)
    for j in range(_RCH):
      pltpu.sync_copy(rows, acc.at[pl.ds(row0 + j * _CHUNK, _CHUNK)])
    if with_cnt:
      def _zc(i, _):
        cntl[pl.ds(i * 16, 16)] = jnp.zeros((16,), jnp.float32)
        return 0
      lax.fori_loop(0, _N_PAD // 16, _zc, 0)
    plsc.subcore_barrier()

    ebase = wid * _EPW
    def _echunk(c, _):
      b = ebase + c * _CHUNK
      pltpu.sync_copy(src_hbm.at[pl.ds(b, _CHUNK)], sidx)
      pltpu.sync_copy(dst_hbm.at[pl.ds(b, _CHUNK)], didx)
      pltpu.async_copy(x_hbm.at[sidx], rows, sem).wait()
      pltpu.sync_copy(rows, acc.at[didx], add=True)
      if with_cnt:
        for j in range(_CHUNK // 16):
          iv = didx[pl.ds(j * 16, 16)]
          plsc.addupdate_scatter(cntl, [iv], jnp.ones((16,), jnp.float32))
      return 0
    lax.fori_loop(0, _NFULL, _echunk, 0)
    bt = ebase + _NFULL * _CHUNK
    pltpu.sync_copy(src_hbm.at[pl.ds(bt, _TAIL)], sidx_t)
    pltpu.sync_copy(dst_hbm.at[pl.ds(bt, _TAIL)], didx_t)
    pltpu.async_copy(x_hbm.at[sidx_t], rows_t, sem).wait()
    pltpu.sync_copy(rows_t, acc.at[didx_t], add=True)
    if with_cnt:
      iv = didx_t[pl.ds(0, _TAIL)]
      plsc.addupdate_scatter(cntl, [iv], jnp.ones((_TAIL,), jnp.float32))
    plsc.subcore_barrier()
    if with_cnt:
      pltpu.sync_copy(cntl, cnt_hbm.at[wid])

    for j in range(_RCH):
      r = row0 + j * _CHUNK
      pltpu.sync_copy(acc.at[pl.ds(r, _CHUNK)], rows)
      pltpu.sync_copy(rows, out_hbm.at[cid, pl.ds(r, _CHUNK)])

  return pl.kernel(body, out_type=out_type, mesh=mesh, scratch_types=scratch,
                   compiler_params=pltpu.CompilerParams(
                       needs_layout_passes=False))


_sc_agg128c = _make_sc_agg(128, with_cnt=True)
_sc_agg128 = _make_sc_agg(128, with_cnt=False)


def _dotT(a, w):
  return lax.dot_general(a, w, (((1,), (1,)), ((), ())),
                         preferred_element_type=jnp.float32)


def _tc_mid_body(agg_ref, cnt_ref, x_ref, w1l_ref, b1l_ref, w1r_ref,
                 w2r_ref, b2l_ref, h_ref, r2_ref):
  agg = agg_ref[0] + agg_ref[1]
  cnt = jnp.sum(cnt_ref[...], axis=1, keepdims=True)
  inv = 1.0 / jnp.maximum(cnt, 1.0)
  mean = agg * inv
  h = _dotT(mean, w1l_ref[...]) + b1l_ref[...] + _dotT(x_ref[...], w1r_ref[...])
  h = jnp.maximum(h, 0.0)
  h_ref[...] = h
  r2_ref[...] = _dotT(h, w2r_ref[...]) + b2l_ref[...]


def _tc_out_body(agg_ref, cnt_ref, r2_ref, w2l_ref, out_ref):
  agg = agg_ref[0] + agg_ref[1]
  cnt = jnp.sum(cnt_ref[...], axis=1, keepdims=True)
  inv = 1.0 / jnp.maximum(cnt, 1.0)
  out_ref[...] = _dotT(agg * inv, w2l_ref[...]) + r2_ref[...]


_B = 1024


def _tc_mid(agg, cnt, x_pad, W1l, b1, W1r, W2r, b2):
  return pl.pallas_call(
      _tc_mid_body,
      grid=(_N_PAD // _B,),
      in_specs=[
          pl.BlockSpec((2, _B, 128), lambda i: (0, i, 0)),
          pl.BlockSpec((_B, _NW), lambda i: (i, 0)),
          pl.BlockSpec((_B, 128), lambda i: (i, 0)),
          pl.BlockSpec((128, 128), lambda i: (0, 0)),
          pl.BlockSpec((1, 128), lambda i: (0, 0)),
          pl.BlockSpec((128, 128), lambda i: (0, 0)),
          pl.BlockSpec((64, 128), lambda i: (0, 0)),
          pl.BlockSpec((1, 64), lambda i: (0, 0)),
      ],
      out_specs=[
          pl.BlockSpec((_B, 128), lambda i: (i, 0)),
          pl.BlockSpec((_B, 64), lambda i: (i, 0)),
      ],
      out_shape=[jax.ShapeDtypeStruct((_N_PAD, 128), jnp.float32),
                 jax.ShapeDtypeStruct((_N_PAD, 64), jnp.float32)],
  )(agg, cnt, x_pad, W1l, b1, W1r, W2r, b2)


def _tc_out(agg2, cnt, r2, W2l):
  return pl.pallas_call(
      _tc_out_body,
      grid=(_N_PAD // _B,),
      in_specs=[
          pl.BlockSpec((2, _B, 128), lambda i: (0, i, 0)),
          pl.BlockSpec((_B, _NW), lambda i: (i, 0)),
          pl.BlockSpec((_B, 64), lambda i: (i, 0)),
          pl.BlockSpec((64, 128), lambda i: (0, 0)),
      ],
      out_specs=pl.BlockSpec((_B, 64), lambda i: (i, 0)),
      out_shape=jax.ShapeDtypeStruct((_N_PAD, 64), jnp.float32),
  )(agg2, cnt, r2, W2l)


def kernel(x, edge_index, W1l, b1l, W1r, W2l, b2l, W2r):
  src = edge_index[0].astype(jnp.int32)
  dst = edge_index[1].astype(jnp.int32)
  x_pad = jnp.pad(x, ((0, _N_PAD - _N_NODES), (0, 0)))
  agg1, cnt = _sc_agg128c(x_pad, src, dst)
  cnt_t = cnt.T
  h, r2 = _tc_mid(agg1, cnt_t, x_pad, W1l, b1l.reshape(1, -1), W1r,
                  W2r, b2l.reshape(1, -1))
  (agg2,) = _sc_agg128(h, src, dst)
  out = _tc_out(agg2, cnt_t, r2, W2l)
  return out[:_N_NODES]


# speedup vs baseline: 10.5711x; 1.5394x over previous
import jax
import jax.numpy as jnp
from jax import lax
from jax.experimental import pallas as pl
from jax.experimental.pallas import tpu as pltpu
from jax.experimental.pallas import tpu_sc as plsc

_N_NODES = 10000
_N_EDGES = 320000
_N_PAD = 10240
_NC, _NS = 2, 16
_NW = _NC * _NS
_CHUNK = 128
_CPW = 80
_E_PAD = _NW * _CPW * _CHUNK
_NBUF = 2
_NPASS = 2
_CPP = _CPW // _NPASS
_NGRP = _CPP // _NBUF
_RPT = _N_PAD // _NS
_RCH = _RPT // _CHUNK


def _make_sc_agg(d):
  mesh = plsc.VectorSubcoreMesh(core_axis_name="c", subcore_axis_name="s")
  out_type = [jax.ShapeDtypeStruct((_NC, _N_PAD, d), jnp.float32)]
  scratch = [
      pltpu.VMEM_SHARED((_N_PAD, d), jnp.float32),
      pltpu.VMEM((_CPP, _CHUNK), jnp.int32),
      pltpu.VMEM((_CPP, _CHUNK), jnp.int32),
      [pltpu.VMEM((_CHUNK, d), jnp.float32) for _ in range(_NBUF)],
      [pltpu.SemaphoreType.DMA for _ in range(_NBUF)],
      [pltpu.SemaphoreType.DMA for _ in range(_NBUF)],
  ]

  def body(x_hbm, src_hbm, dst_hbm, out_hbm, acc, sidx, didx, rows, gsem,
           ssem):
    cid = lax.axis_index("c")
    sid = lax.axis_index("s")
    wid = sid * _NC + cid
    row0 = sid * _RPT

    def _zrow(i, _):
      for j in range(d // 16):
        rows[0][i, pl.ds(j * 16, 16)] = jnp.zeros((16,), jnp.float32)
      return 0
    lax.fori_loop(0, _CHUNK, _zrow, 0)
    for j in range(_RCH):
      pltpu.sync_copy(rows[0], acc.at[pl.ds(row0 + j * _CHUNK, _CHUNK)])
    plsc.subcore_barrier()

    for p in range(_NPASS):
      base = wid * _CPW + p * _CPP
      pltpu.sync_copy(src_hbm.at[pl.ds(base, _CPP)], sidx)
      pltpu.sync_copy(dst_hbm.at[pl.ds(base, _CPP)], didx)
      for b in range(_NBUF):
        pltpu.async_copy(x_hbm.at[sidx.at[b]], rows[b], gsem[b])

      def _group(g, _):
        for b in range(_NBUF):
          j = g * _NBUF + b
          pltpu.make_async_copy(x_hbm.at[sidx.at[0]], rows[b],
                                gsem[b]).wait()
          pltpu.async_copy(rows[b], acc.at[didx.at[j]], ssem[b], add=True)
        for b in range(_NBUF):
          pltpu.make_async_copy(rows[b], acc.at[didx.at[0]], ssem[b]).wait()
          pltpu.async_copy(x_hbm.at[sidx.at[(g + 1) * _NBUF + b]], rows[b],
                           gsem[b])
        return 0
      lax.fori_loop(0, _NGRP - 1, _group, 0)
      for b in range(_NBUF):
        j = (_NGRP - 1) * _NBUF + b
        pltpu.make_async_copy(x_hbm.at[sidx.at[0]], rows[b], gsem[b]).wait()
        pltpu.async_copy(rows[b], acc.at[didx.at[j]], ssem[b], add=True)
      for b in range(_NBUF):
        pltpu.make_async_copy(rows[b], acc.at[didx.at[0]], ssem[b]).wait()
    plsc.subcore_barrier()

    for j in range(_RCH):
      r = row0 + j * _CHUNK
      pltpu.sync_copy(acc.at[pl.ds(r, _CHUNK)], rows[0])
      pltpu.sync_copy(rows[0], out_hbm.at[cid, pl.ds(r, _CHUNK)])

  return pl.kernel(body, out_type=out_type, mesh=mesh, scratch_types=scratch,
                   compiler_params=pltpu.CompilerParams(
                       needs_layout_passes=False))


def _make_sc_cnt():
  mesh = plsc.VectorSubcoreMesh(core_axis_name="c", subcore_axis_name="s")

  def body(dst_hbm, cnt_hbm, didx, cntl):
    cid = lax.axis_index("c")
    sid = lax.axis_index("s")
    wid = sid * _NC + cid
    def _zc(i, _):
      cntl[pl.ds(i * 16, 16)] = jnp.zeros((16,), jnp.float32)
      return 0
    lax.fori_loop(0, _N_PAD // 16, _zc, 0)
    pltpu.sync_copy(dst_hbm.at[pl.ds(wid * _CPW, _CPW)], didx)
    ones16 = jnp.ones((16,), jnp.float32)
    def _chunk(j, _):
      for k in range(_CHUNK // 16):
        iv = didx[j, pl.ds(k * 16, 16)]
        plsc.addupdate_scatter(cntl, [iv], ones16)
      return 0
    lax.fori_loop(0, _CPW, _chunk, 0)
    pltpu.sync_copy(cntl, cnt_hbm.at[wid])

  return pl.kernel(
      body,
      out_type=[jax.ShapeDtypeStruct((_NW, _N_PAD), jnp.float32)],
      mesh=mesh,
      scratch_types=[pltpu.VMEM((_CPW, _CHUNK), jnp.int32),
                     pltpu.VMEM((_N_PAD,), jnp.float32)],
      compiler_params=pltpu.CompilerParams(needs_layout_passes=False))


_sc_agg128 = _make_sc_agg(128)
_sc_cnt = _make_sc_cnt()


def _dotT(a, w):
  return lax.dot_general(a, w, (((1,), (1,)), ((), ())),
                         preferred_element_type=jnp.float32)


def _tc_mid_body(agg_ref, cnt_ref, x_ref, w1l_ref, b1l_ref, w1r_ref,
                 w2r_ref, b2l_ref, h_ref, r2_ref):
  agg = agg_ref[0] + agg_ref[1]
  cnt = jnp.sum(cnt_ref[...], axis=1, keepdims=True)
  inv = 1.0 / jnp.maximum(cnt, 1.0)
  mean = agg * inv
  h = _dotT(mean, w1l_ref[...]) + b1l_ref[...] + _dotT(x_ref[...], w1r_ref[...])
  h = jnp.maximum(h, 0.0)
  h_ref[...] = h
  r2_ref[...] = _dotT(h, w2r_ref[...]) + b2l_ref[...]


def _tc_out_body(agg_ref, cnt_ref, r2_ref, w2l_ref, out_ref):
  agg = agg_ref[0] + agg_ref[1]
  cnt = jnp.sum(cnt_ref[...], axis=1, keepdims=True)
  inv = 1.0 / jnp.maximum(cnt, 1.0)
  out_ref[...] = _dotT(agg * inv, w2l_ref[...]) + r2_ref[...]


_B = 1024


def _tc_mid(agg, cnt, x_pad, W1l, b1, W1r, W2r, b2):
  return pl.pallas_call(
      _tc_mid_body,
      grid=(_N_PAD // _B,),
      in_specs=[
          pl.BlockSpec((2, _B, 128), lambda i: (0, i, 0)),
          pl.BlockSpec((_B, _NW), lambda i: (i, 0)),
          pl.BlockSpec((_B, 128), lambda i: (i, 0)),
          pl.BlockSpec((128, 128), lambda i: (0, 0)),
          pl.BlockSpec((1, 128), lambda i: (0, 0)),
          pl.BlockSpec((128, 128), lambda i: (0, 0)),
          pl.BlockSpec((64, 128), lambda i: (0, 0)),
          pl.BlockSpec((1, 64), lambda i: (0, 0)),
      ],
      out_specs=[
          pl.BlockSpec((_B, 128), lambda i: (i, 0)),
          pl.BlockSpec((_B, 64), lambda i: (i, 0)),
      ],
      out_shape=[jax.ShapeDtypeStruct((_N_PAD, 128), jnp.float32),
                 jax.ShapeDtypeStruct((_N_PAD, 64), jnp.float32)],
  )(agg, cnt, x_pad, W1l, b1, W1r, W2r, b2)


def _tc_out(agg2, cnt, r2, W2l):
  return pl.pallas_call(
      _tc_out_body,
      grid=(_N_PAD // _B,),
      in_specs=[
          pl.BlockSpec((2, _B, 128), lambda i: (0, i, 0)),
          pl.BlockSpec((_B, _NW), lambda i: (i, 0)),
          pl.BlockSpec((_B, 64), lambda i: (i, 0)),
          pl.BlockSpec((64, 128), lambda i: (0, 0)),
      ],
      out_specs=pl.BlockSpec((_B, 64), lambda i: (i, 0)),
      out_shape=jax.ShapeDtypeStruct((_N_PAD, 64), jnp.float32),
  )(agg2, cnt, r2, W2l)


def kernel(x, edge_index, W1l, b1l, W1r, W2l, b2l, W2r):
  ei = edge_index.astype(jnp.int32)
  pad = jnp.broadcast_to(
      _N_NODES + (jnp.arange(_E_PAD - _N_EDGES, dtype=jnp.int32)
                  % (_N_PAD - _N_NODES)), (2, _E_PAD - _N_EDGES))
  ei = jnp.concatenate([ei, pad], axis=1)
  src = ei[0].reshape(_NW * _CPW, _CHUNK)
  dst = ei[1].reshape(_NW * _CPW, _CHUNK)
  x_pad = jnp.pad(x, ((0, _N_PAD - _N_NODES), (0, 0)))
  (agg1,) = _sc_agg128(x_pad, src, dst)
  (cnt,) = _sc_cnt(dst)
  cnt_t = cnt.T
  h, r2 = _tc_mid(agg1, cnt_t, x_pad, W1l, b1l.reshape(1, -1), W1r,
                  W2r, b2l.reshape(1, -1))
  (agg2,) = _sc_agg128(h, src, dst)
  out = _tc_out(agg2, cnt_t, r2, W2l)
  return out[:_N_NODES]


# speedup vs baseline: 11.7004x; 1.1068x over previous
import jax
import jax.numpy as jnp
from jax import lax
from jax.experimental import pallas as pl
from jax.experimental.pallas import tpu as pltpu
from jax.experimental.pallas import tpu_sc as plsc

_N_NODES = 10000
_N_EDGES = 320000
_N_PAD = 10240
_NC, _NS = 2, 16
_NW = _NC * _NS
_CHUNK = 64
_CPW = 160
_E_PAD = _NW * _CPW * _CHUNK
_NBUF = 4
_NPASS = 4
_CPP = _CPW // _NPASS
_NGRP = _CPP // _NBUF
_RPT = _N_PAD // _NS
_RCH = _RPT // _CHUNK


def _make_sc_agg(d):
  mesh = plsc.VectorSubcoreMesh(core_axis_name="c", subcore_axis_name="s")
  out_type = [jax.ShapeDtypeStruct((_NC, _N_PAD, d), jnp.float32)]
  scratch = [
      pltpu.VMEM_SHARED((_N_PAD, d), jnp.float32),
      pltpu.VMEM((_CPP, _CHUNK), jnp.int32),
      pltpu.VMEM((_CPP, _CHUNK), jnp.int32),
      [pltpu.VMEM((_CHUNK, d), jnp.float32) for _ in range(_NBUF)],
      [pltpu.SemaphoreType.DMA for _ in range(_NBUF)],
      [pltpu.SemaphoreType.DMA for _ in range(_NBUF)],
  ]

  def body(x_hbm, src_hbm, dst_hbm, out_hbm, acc, sidx, didx, rows, gsem,
           ssem):
    cid = lax.axis_index("c")
    sid = lax.axis_index("s")
    wid = sid * _NC + cid
    row0 = sid * _RPT

    def _zrow(i, _):
      for j in range(d // 16):
        rows[0][i, pl.ds(j * 16, 16)] = jnp.zeros((16,), jnp.float32)
      return 0
    lax.fori_loop(0, _CHUNK, _zrow, 0)
    for j in range(_RCH):
      pltpu.sync_copy(rows[0], acc.at[pl.ds(row0 + j * _CHUNK, _CHUNK)])
    plsc.subcore_barrier()

    for p in range(_NPASS):
      base = wid * _CPW + p * _CPP
      pltpu.sync_copy(src_hbm.at[pl.ds(base, _CPP)], sidx)
      pltpu.sync_copy(dst_hbm.at[pl.ds(base, _CPP)], didx)
      for b in range(_NBUF):
        pltpu.async_copy(x_hbm.at[sidx.at[b]], rows[b], gsem[b])

      def _group(g, _):
        for b in range(_NBUF):
          j = g * _NBUF + b
          pltpu.make_async_copy(x_hbm.at[sidx.at[0]], rows[b],
                                gsem[b]).wait()
          pltpu.async_copy(rows[b], acc.at[didx.at[j]], ssem[b], add=True)
        for b in range(_NBUF):
          pltpu.make_async_copy(rows[b], acc.at[didx.at[0]], ssem[b]).wait()
          pltpu.async_copy(x_hbm.at[sidx.at[(g + 1) * _NBUF + b]], rows[b],
                           gsem[b])
        return 0
      lax.fori_loop(0, _NGRP - 1, _group, 0)
      for b in range(_NBUF):
        j = (_NGRP - 1) * _NBUF + b
        pltpu.make_async_copy(x_hbm.at[sidx.at[0]], rows[b], gsem[b]).wait()
        pltpu.async_copy(rows[b], acc.at[didx.at[j]], ssem[b], add=True)
      for b in range(_NBUF):
        pltpu.make_async_copy(rows[b], acc.at[didx.at[0]], ssem[b]).wait()
    plsc.subcore_barrier()

    for j in range(_RCH):
      r = row0 + j * _CHUNK
      pltpu.sync_copy(acc.at[pl.ds(r, _CHUNK)], rows[0])
      pltpu.sync_copy(rows[0], out_hbm.at[cid, pl.ds(r, _CHUNK)])

  return pl.kernel(body, out_type=out_type, mesh=mesh, scratch_types=scratch,
                   compiler_params=pltpu.CompilerParams(
                       needs_layout_passes=False))


def _make_sc_cnt():
  mesh = plsc.VectorSubcoreMesh(core_axis_name="c", subcore_axis_name="s")

  def body(dst_hbm, cnt_hbm, didx, cntl):
    cid = lax.axis_index("c")
    sid = lax.axis_index("s")
    wid = sid * _NC + cid
    def _zc(i, _):
      cntl[pl.ds(i * 16, 16)] = jnp.zeros((16,), jnp.float32)
      return 0
    lax.fori_loop(0, _N_PAD // 16, _zc, 0)
    pltpu.sync_copy(dst_hbm.at[pl.ds(wid * _CPW, _CPW)], didx)
    ones16 = jnp.ones((16,), jnp.float32)
    def _chunk(j, _):
      for k in range(_CHUNK // 16):
        iv = didx[j, pl.ds(k * 16, 16)]
        plsc.addupdate_scatter(cntl, [iv], ones16)
      return 0
    lax.fori_loop(0, _CPW, _chunk, 0)
    pltpu.sync_copy(cntl, cnt_hbm.at[wid])

  return pl.kernel(
      body,
      out_type=[jax.ShapeDtypeStruct((_NW, _N_PAD), jnp.float32)],
      mesh=mesh,
      scratch_types=[pltpu.VMEM((_CPW, _CHUNK), jnp.int32),
                     pltpu.VMEM((_N_PAD,), jnp.float32)],
      compiler_params=pltpu.CompilerParams(needs_layout_passes=False))


_sc_agg128 = _make_sc_agg(128)
_sc_cnt = _make_sc_cnt()


def _dotT(a, w):
  return lax.dot_general(a, w, (((1,), (1,)), ((), ())),
                         preferred_element_type=jnp.float32)


def _tc_mid_body(agg_ref, cnt_ref, x_ref, w1l_ref, b1l_ref, w1r_ref,
                 w2r_ref, b2l_ref, h_ref, r2_ref):
  agg = agg_ref[0] + agg_ref[1]
  cnt = jnp.sum(cnt_ref[...], axis=1, keepdims=True)
  inv = 1.0 / jnp.maximum(cnt, 1.0)
  mean = agg * inv
  h = _dotT(mean, w1l_ref[...]) + b1l_ref[...] + _dotT(x_ref[...], w1r_ref[...])
  h = jnp.maximum(h, 0.0)
  h_ref[...] = h
  r2_ref[...] = _dotT(h, w2r_ref[...]) + b2l_ref[...]


def _tc_out_body(agg_ref, cnt_ref, r2_ref, w2l_ref, out_ref):
  agg = agg_ref[0] + agg_ref[1]
  cnt = jnp.sum(cnt_ref[...], axis=1, keepdims=True)
  inv = 1.0 / jnp.maximum(cnt, 1.0)
  out_ref[...] = _dotT(agg * inv, w2l_ref[...]) + r2_ref[...]


_B = 1024


def _tc_mid(agg, cnt, x_pad, W1l, b1, W1r, W2r, b2):
  return pl.pallas_call(
      _tc_mid_body,
      grid=(_N_PAD // _B,),
      in_specs=[
          pl.BlockSpec((2, _B, 128), lambda i: (0, i, 0)),
          pl.BlockSpec((_B, _NW), lambda i: (i, 0)),
          pl.BlockSpec((_B, 128), lambda i: (i, 0)),
          pl.BlockSpec((128, 128), lambda i: (0, 0)),
          pl.BlockSpec((1, 128), lambda i: (0, 0)),
          pl.BlockSpec((128, 128), lambda i: (0, 0)),
          pl.BlockSpec((64, 128), lambda i: (0, 0)),
          pl.BlockSpec((1, 64), lambda i: (0, 0)),
      ],
      out_specs=[
          pl.BlockSpec((_B, 128), lambda i: (i, 0)),
          pl.BlockSpec((_B, 64), lambda i: (i, 0)),
      ],
      out_shape=[jax.ShapeDtypeStruct((_N_PAD, 128), jnp.float32),
                 jax.ShapeDtypeStruct((_N_PAD, 64), jnp.float32)],
  )(agg, cnt, x_pad, W1l, b1, W1r, W2r, b2)


def _tc_out(agg2, cnt, r2, W2l):
  return pl.pallas_call(
      _tc_out_body,
      grid=(_N_PAD // _B,),
      in_specs=[
          pl.BlockSpec((2, _B, 128), lambda i: (0, i, 0)),
          pl.BlockSpec((_B, _NW), lambda i: (i, 0)),
          pl.BlockSpec((_B, 64), lambda i: (i, 0)),
          pl.BlockSpec((64, 128), lambda i: (0, 0)),
      ],
      out_specs=pl.BlockSpec((_B, 64), lambda i: (i, 0)),
      out_shape=jax.ShapeDtypeStruct((_N_PAD, 64), jnp.float32),
  )(agg2, cnt, r2, W2l)


def kernel(x, edge_index, W1l, b1l, W1r, W2l, b2l, W2r):
  ei = edge_index.astype(jnp.int32)
  pad = jnp.broadcast_to(
      _N_NODES + (jnp.arange(_E_PAD - _N_EDGES, dtype=jnp.int32)
                  % (_N_PAD - _N_NODES)), (2, _E_PAD - _N_EDGES))
  ei = jnp.concatenate([ei, pad], axis=1)
  src = ei[0].reshape(_NW * _CPW, _CHUNK)
  dst = ei[1].reshape(_NW * _CPW, _CHUNK)
  x_pad = jnp.pad(x, ((0, _N_PAD - _N_NODES), (0, 0)))
  (agg1,) = _sc_agg128(x_pad, src, dst)
  (cnt,) = _sc_cnt(dst)
  cnt_t = cnt.T
  h, r2 = _tc_mid(agg1, cnt_t, x_pad, W1l, b1l.reshape(1, -1), W1r,
                  W2r, b2l.reshape(1, -1))
  (agg2,) = _sc_agg128(h, src, dst)
  out = _tc_out(agg2, cnt_t, r2, W2l)
  return out[:_N_NODES]
